# baseline (speedup 1.0000x reference)
import functools

import jax
import jax.numpy as jnp
from jax.experimental import pallas as pl
from jax.experimental.pallas import tpu as pltpu
from jax.sharding import Mesh, PartitionSpec as P

N = 10000
F = 128
BM = 200


def _s1_kernel(x_ref, w1_ref, s1_ref):
    s1_ref[...] = jnp.dot(x_ref[...], w1_ref[...],
                          preferred_element_type=jnp.float32
                          ).astype(jnp.bfloat16)


def _layer1_kernel(adj_ref, s1_ref, b1_ref, w2_ref, h_ref, s2_ref):
    p = jnp.dot(adj_ref[...].astype(jnp.bfloat16), s1_ref[...],
                preferred_element_type=jnp.float32)
    h = jnp.maximum(p + b1_ref[...], 0.0)
    h_ref[...] = h
    s2_ref[...] = jnp.dot(h.astype(jnp.bfloat16), w2_ref[...],
                          preferred_element_type=jnp.float32
                          ).astype(jnp.bfloat16)


def _layer2_kernel(adj_ref, s2_ref, b2_ref, h_ref, out_ref):
    p = jnp.dot(adj_ref[...].astype(jnp.bfloat16), s2_ref[...],
                preferred_element_type=jnp.float32)
    out_ref[...] = jnp.maximum(p + b2_ref[...], 0.0) + h_ref[...]


def _gcn2_local(x, adj, W1, b1r, b2r, w2b, axis_name):
    rows = adj.shape[0]

    s1 = pl.pallas_call(
        _s1_kernel,
        grid=(N // 1000,),
        in_specs=[
            pl.BlockSpec((1000, F), lambda m: (m, 0)),
            pl.BlockSpec((F, F), lambda m: (0, 0)),
        ],
        out_specs=pl.BlockSpec((1000, F), lambda m: (m, 0)),
        out_shape=jax.ShapeDtypeStruct((N, F), jnp.bfloat16),
        compiler_params=pltpu.CompilerParams(
            dimension_semantics=("parallel",)),
    )(x, W1)

    h, s2 = pl.pallas_call(
        _layer1_kernel,
        grid=(rows // BM,),
        in_specs=[
            pl.BlockSpec((BM, N), lambda m: (m, 0)),
            pl.BlockSpec((N, F), lambda m: (0, 0)),
            pl.BlockSpec((1, F), lambda m: (0, 0)),
            pl.BlockSpec((F, F), lambda m: (0, 0)),
        ],
        out_specs=[
            pl.BlockSpec((BM, F), lambda m: (m, 0)),
            pl.BlockSpec((BM, F), lambda m: (m, 0)),
        ],
        out_shape=[
            jax.ShapeDtypeStruct((rows, F), jnp.float32),
            jax.ShapeDtypeStruct((rows, F), jnp.bfloat16),
        ],
        compiler_params=pltpu.CompilerParams(
            dimension_semantics=("parallel",)),
    )(adj, s1, b1r, w2b)

    if axis_name is not None:
        s2_full = jax.lax.all_gather(s2, axis_name, axis=0, tiled=True)
    else:
        s2_full = s2

    out = pl.pallas_call(
        _layer2_kernel,
        grid=(rows // BM,),
        in_specs=[
            pl.BlockSpec((BM, N), lambda m: (m, 0)),
            pl.BlockSpec((N, F), lambda m: (0, 0)),
            pl.BlockSpec((1, F), lambda m: (0, 0)),
            pl.BlockSpec((BM, F), lambda m: (m, 0)),
        ],
        out_specs=pl.BlockSpec((BM, F), lambda m: (m, 0)),
        out_shape=jax.ShapeDtypeStruct((rows, F), jnp.float32),
        compiler_params=pltpu.CompilerParams(
            dimension_semantics=("parallel",)),
    )(adj, s2_full, b2r, h)

    return out


def kernel(x, adj, W1, b1, W2, b2):
    b1r = b1.reshape(1, F)
    b2r = b2.reshape(1, F)
    w2b = W2.astype(jnp.bfloat16)

    devs = jax.devices()
    n_shards = 2 if (len(devs) >= 2 and N % (2 * BM) == 0) else 1

    if n_shards == 1:
        return _gcn2_local(x, adj, W1, b1r, b2r, w2b, None)

    mesh = Mesh(devs[:n_shards], ("i",))
    fn = jax.shard_map(
        functools.partial(_gcn2_local, axis_name="i"),
        mesh=mesh,
        in_specs=(P(None, None), P("i", None), P(None, None),
                  P(None, None), P(None, None), P(None, None)),
        out_specs=P("i", None),
        check_vma=False,
    )
    return fn(x, adj, W1, b1r, b2r, w2b)


# speedup vs baseline: 3.0146x; 3.0146x over previous
import jax
import jax.numpy as jnp
from jax.experimental import pallas as pl
from jax.experimental.pallas import tpu as pltpu

N = 10000
F = 128
BM = 200
NB = N // BM


def _gcn2_kernel(adj_ref, x_ref, w1_ref, w2_ref, b1_ref, b2_ref,
                 out_ref, s1_ref, h_ref, s2_ref):
    m = pl.program_id(0)
    r = (m % NB) * BM

    @pl.when(m == 0)
    def _():
        s1_ref[...] = jnp.dot(x_ref[...], w1_ref[...],
                              preferred_element_type=jnp.float32
                              ).astype(jnp.bfloat16)

    a = adj_ref[...].astype(jnp.bfloat16)

    @pl.when(m < NB)
    def _():
        p = jnp.dot(a, s1_ref[...], preferred_element_type=jnp.float32)
        h = jnp.maximum(p + b1_ref[...], 0.0)
        h_ref[pl.ds(r, BM), :] = h
        s2_ref[pl.ds(r, BM), :] = jnp.dot(
            h.astype(jnp.bfloat16), w2_ref[...],
            preferred_element_type=jnp.float32).astype(jnp.bfloat16)

    @pl.when(m >= NB)
    def _():
        p = jnp.dot(a, s2_ref[...], preferred_element_type=jnp.float32)
        out_ref[...] = (jnp.maximum(p + b2_ref[...], 0.0)
                        + h_ref[pl.ds(r, BM), :])


def kernel(x, adj, W1, b1, W2, b2):
    b1r = b1.reshape(1, F)
    b2r = b2.reshape(1, F)
    w2b = W2.astype(jnp.bfloat16)

    out = pl.pallas_call(
        _gcn2_kernel,
        grid=(2 * NB,),
        in_specs=[
            pl.BlockSpec((BM, N), lambda m: (m % NB, 0)),
            pl.BlockSpec((N, F), lambda m: (0, 0)),
            pl.BlockSpec((F, F), lambda m: (0, 0)),
            pl.BlockSpec((F, F), lambda m: (0, 0)),
            pl.BlockSpec((1, F), lambda m: (0, 0)),
            pl.BlockSpec((1, F), lambda m: (0, 0)),
        ],
        out_specs=pl.BlockSpec((BM, F), lambda m: (jnp.maximum(m - NB, 0), 0)),
        out_shape=jax.ShapeDtypeStruct((N, F), jnp.float32),
        scratch_shapes=[
            pltpu.VMEM((N, F), jnp.bfloat16),
            pltpu.VMEM((N, F), jnp.float32),
            pltpu.VMEM((N, F), jnp.bfloat16),
        ],
        compiler_params=pltpu.CompilerParams(
            dimension_semantics=("arbitrary",)),
    )(adj, x, W1, w2b, b1r, b2r)

    return out


# speedup vs baseline: 3.3820x; 1.1219x over previous
import jax
import jax.numpy as jnp
from jax.experimental import pallas as pl
from jax.experimental.pallas import tpu as pltpu

N = 10000
F = 128
BM = 400
NB = N // BM


def _gcn2_kernel(adj_ref, x_ref, w1_ref, w2_ref, b1_ref, b2_ref,
                 out_ref, s1_ref, h_ref, s2_ref):
    m = pl.program_id(0)
    r = (m % NB) * BM

    @pl.when(m == 0)
    def _():
        s1_ref[...] = jnp.dot(x_ref[...], w1_ref[...],
                              preferred_element_type=jnp.float32)

    @pl.when(m < NB)
    def _():
        p = jnp.dot(adj_ref[...], s1_ref[...],
                    preferred_element_type=jnp.float32)
        h = jnp.maximum(p + b1_ref[...], 0.0)
        h_ref[pl.ds(r, BM), :] = h
        s2_ref[pl.ds(r, BM), :] = jnp.dot(h, w2_ref[...],
                                          preferred_element_type=jnp.float32)

    @pl.when(m >= NB)
    def _():
        p = jnp.dot(adj_ref[...], s2_ref[...],
                    preferred_element_type=jnp.float32)
        out_ref[...] = (jnp.maximum(p + b2_ref[...], 0.0)
                        + h_ref[pl.ds(r, BM), :])


def kernel(x, adj, W1, b1, W2, b2):
    b1r = b1.reshape(1, F)
    b2r = b2.reshape(1, F)

    out = pl.pallas_call(
        _gcn2_kernel,
        grid=(2 * NB,),
        in_specs=[
            pl.BlockSpec((BM, N), lambda m: (m % NB, 0)),
            pl.BlockSpec((N, F), lambda m: (0, 0)),
            pl.BlockSpec((F, F), lambda m: (0, 0)),
            pl.BlockSpec((F, F), lambda m: (0, 0)),
            pl.BlockSpec((1, F), lambda m: (0, 0)),
            pl.BlockSpec((1, F), lambda m: (0, 0)),
        ],
        out_specs=pl.BlockSpec((BM, F), lambda m: (jnp.maximum(m - NB, 0), 0)),
        out_shape=jax.ShapeDtypeStruct((N, F), jnp.float32),
        scratch_shapes=[
            pltpu.VMEM((N, F), jnp.float32),
            pltpu.VMEM((N, F), jnp.float32),
            pltpu.VMEM((N, F), jnp.float32),
        ],
        compiler_params=pltpu.CompilerParams(
            dimension_semantics=("arbitrary",)),
    )(adj, x, W1, W2, b1r, b2r)

    return out
